# Initial kernel scaffold; baseline (speedup 1.0000x reference)
import jax
import jax.numpy as jnp
from jax.experimental import pallas as pl


def kernel(inputs, embeddings, W1, a_src1, a_dst1, b1, W2, a_src2, a_dst2, b2):
    raise NotImplementedError("write your pallas kernel here")


# speedup vs baseline: 6031.9230x; 6031.9230x over previous
import jax
import jax.numpy as jnp
from jax.experimental import pallas as pl

_N = 1000
_NP = 1024
_F_IN = 64
_UNITS = 32
_HEADS = 4


def _fused_gat_kernel(x_ref, w1_ref, ab1_ref, b1_ref, w2_ref, ab2_ref,
                      b2_ref, out_ref):
    x = x_ref[...]
    h1 = jnp.dot(x, w1_ref[...],
                 preferred_element_type=jnp.float32)
    s1 = jnp.dot(h1, ab1_ref[...],
                 preferred_element_type=jnp.float32)
    s1t = s1.T

    col = jax.lax.broadcasted_iota(jnp.int32, (1, _NP), 1)
    colmask = (col < _N).astype(jnp.float32)
    neg_pad = (colmask - 1.0) * 1e30

    def _attend(st_col, ss_row, v):
        e = st_col + ss_row
        e = jnp.where(e >= 0.0, e, 0.2 * e)
        m = jnp.max(e + neg_pad, axis=1, keepdims=True)
        ex = jnp.exp(e - m) * colmask
        den = jnp.sum(ex, axis=1, keepdims=True) + 1e-9
        return jnp.dot(ex / den, v, preferred_element_type=jnp.float32)

    outs = []
    for h in range(_HEADS):
        outs.append(_attend(s1[:, h:h + 1],
                            s1t[_HEADS + h:_HEADS + h + 1, :],
                            h1[:, h * _UNITS:(h + 1) * _UNITS]))
    h1o = jnp.concatenate(outs, axis=1) + b1_ref[...]
    h1o = jnp.maximum(h1o, 0.0)

    h2 = jnp.dot(h1o, w2_ref[...],
                 preferred_element_type=jnp.float32)
    s2 = jnp.dot(h2, ab2_ref[...],
                 preferred_element_type=jnp.float32)
    s2t = s2.T
    out = _attend(s2[:, 0:1], s2t[1:2, :], h2)
    out_ref[...] = out + b2_ref[...]


def kernel(inputs, embeddings, W1, a_src1, a_dst1, b1, W2, a_src2, a_dst2,
           b2):
    del embeddings
    x = jnp.zeros((_NP, _F_IN), jnp.float32).at[:_N].set(inputs)
    w1 = W1.reshape(_F_IN, _HEADS * _UNITS)
    eye = jnp.eye(_HEADS, dtype=jnp.float32)
    ab_src = (eye[:, None, :] * a_src1[:, :, None]).reshape(
        _HEADS * _UNITS, _HEADS)
    ab_dst = (eye[:, None, :] * a_dst1[:, :, None]).reshape(
        _HEADS * _UNITS, _HEADS)
    ab1 = jnp.concatenate([ab_src, ab_dst], axis=1)
    w2 = W2.reshape(_HEADS * _UNITS, _UNITS)
    ab2 = jnp.concatenate([a_src2.T, a_dst2.T], axis=1)

    out = pl.pallas_call(
        _fused_gat_kernel,
        out_shape=jax.ShapeDtypeStruct((_NP, _UNITS), jnp.float32),
    )(x, w1, ab1, b1[None, :], w2, ab2, b2[None, :])
    return out[:_N]


# speedup vs baseline: 7454.7349x; 1.2359x over previous
import jax
import jax.numpy as jnp
from jax.experimental import pallas as pl

_N = 1000
_NP = 1024
_F_IN = 64
_UNITS = 32
_HEADS = 4


_LOG2E = 1.4426950408889634


def _fused_gat_kernel(x_ref, w1_ref, ab1_ref, b1_ref, w2_ref, ab2_ref,
                      b2_ref, out_ref):
    x = x_ref[...]
    h1 = jnp.dot(x, w1_ref[...],
                 preferred_element_type=jnp.float32)
    s1 = jnp.dot(h1, ab1_ref[...],
                 preferred_element_type=jnp.float32) * _LOG2E
    s1t = s1.T

    col = jax.lax.broadcasted_iota(jnp.int32, (1, _NP), 1)
    colmask = (col < _N).astype(jnp.float32)
    neg_pad = (colmask - 1.0) * 1e30
    ones_col = jnp.full((_NP, 1), 1.0, jnp.float32)

    def _attend(st_col, ss_row, v):
        ss_m = ss_row + neg_pad
        ssmax = jnp.max(ss_m, axis=1, keepdims=True)
        m = st_col + ssmax
        m = jnp.maximum(m, 0.2 * m)
        e = st_col + ss_m
        e = jnp.maximum(e, 0.2 * e)
        ex = jnp.exp2(e - m)
        va = jnp.concatenate([v, ones_col], axis=1)
        agg = jnp.dot(ex, va, preferred_element_type=jnp.float32)
        return agg[:, :_UNITS] / (agg[:, _UNITS:_UNITS + 1] + 1e-9)

    outs = []
    for h in range(_HEADS):
        outs.append(_attend(s1[:, h:h + 1],
                            s1t[_HEADS + h:_HEADS + h + 1, :],
                            h1[:, h * _UNITS:(h + 1) * _UNITS]))
    h1o = jnp.concatenate(outs, axis=1) + b1_ref[...]
    h1o = jnp.maximum(h1o, 0.0)

    h2 = jnp.dot(h1o, w2_ref[...],
                 preferred_element_type=jnp.float32)
    s2 = jnp.dot(h2, ab2_ref[...],
                 preferred_element_type=jnp.float32) * _LOG2E
    s2t = s2.T
    out = _attend(s2[:, 0:1], s2t[1:2, :], h2)
    out_ref[...] = out + b2_ref[...]


def kernel(inputs, embeddings, W1, a_src1, a_dst1, b1, W2, a_src2, a_dst2,
           b2):
    del embeddings
    x = jnp.zeros((_NP, _F_IN), jnp.float32).at[:_N].set(inputs)
    w1 = W1.reshape(_F_IN, _HEADS * _UNITS)
    eye = jnp.eye(_HEADS, dtype=jnp.float32)
    ab_src = (eye[:, None, :] * a_src1[:, :, None]).reshape(
        _HEADS * _UNITS, _HEADS)
    ab_dst = (eye[:, None, :] * a_dst1[:, :, None]).reshape(
        _HEADS * _UNITS, _HEADS)
    ab1 = jnp.concatenate([ab_src, ab_dst], axis=1)
    w2 = W2.reshape(_HEADS * _UNITS, _UNITS)
    ab2 = jnp.concatenate([a_src2.T, a_dst2.T], axis=1)

    out = pl.pallas_call(
        _fused_gat_kernel,
        out_shape=jax.ShapeDtypeStruct((_NP, _UNITS), jnp.float32),
    )(x, w1, ab1, b1[None, :], w2, ab2, b2[None, :])
    return out[:_N]


# speedup vs baseline: 7675.1002x; 1.0296x over previous
import jax
import jax.numpy as jnp
from jax.experimental import pallas as pl

_N = 1000
_NP = 1024
_F_IN = 64
_UNITS = 32
_HEADS = 4
_LOG2E = 1.4426950408889634


def _head_mix(nrows):
    r = jax.lax.broadcasted_iota(jnp.int32, (nrows, nrows // _UNITS), 0)
    c = jax.lax.broadcasted_iota(jnp.int32, (nrows, nrows // _UNITS), 1)
    return jnp.where(r // _UNITS == c, _LOG2E, 0.0).astype(jnp.float32)


def _fused_gat_kernel(x_ref, w1_ref, as1_ref, ad1_ref, b1_ref, w2_ref,
                      as2_ref, ad2_ref, b2_ref, out_ref):
    x = jnp.concatenate(
        [x_ref[...], jnp.zeros((_NP - _N, _F_IN), jnp.float32)], axis=0)
    h1 = jnp.dot(x, w1_ref[...],
                 preferred_element_type=jnp.float32)
    s1 = jnp.dot(
        jnp.concatenate([h1 * as1_ref[...], h1 * ad1_ref[...]], axis=1),
        _head_mix(2 * _HEADS * _UNITS),
        preferred_element_type=jnp.float32)
    s1t = s1.T

    col = jax.lax.broadcasted_iota(jnp.int32, (1, _NP), 1)
    neg_pad = jnp.where(col < _N, 0.0, -1e30).astype(jnp.float32)
    ones_col = jnp.full((_NP, 1), 1.0, jnp.float32)

    def _attend(st_col, ss_row, v):
        ss_m = ss_row + neg_pad
        ssmax = jnp.max(ss_m, axis=1, keepdims=True)
        m = st_col + ssmax
        m = jnp.maximum(m, 0.2 * m)
        e = st_col + ss_m
        e = jnp.maximum(e, 0.2 * e)
        ex = jnp.exp2(e - m)
        va = jnp.concatenate([v, ones_col], axis=1)
        agg = jnp.dot(ex, va, preferred_element_type=jnp.float32)
        return agg[:, :_UNITS] / (agg[:, _UNITS:_UNITS + 1] + 1e-9)

    outs = []
    for h in range(_HEADS):
        outs.append(_attend(s1[:, h:h + 1],
                            s1t[_HEADS + h:_HEADS + h + 1, :],
                            h1[:, h * _UNITS:(h + 1) * _UNITS]))
    h1o = jnp.concatenate(outs, axis=1) + b1_ref[...]
    h1o = jnp.maximum(h1o, 0.0)

    h2 = jnp.dot(h1o, w2_ref[...],
                 preferred_element_type=jnp.float32)
    s2 = jnp.dot(
        jnp.concatenate([h2 * as2_ref[...], h2 * ad2_ref[...]], axis=1),
        _head_mix(2 * _UNITS),
        preferred_element_type=jnp.float32)
    s2t = s2.T
    out = _attend(s2[:, 0:1], s2t[1:2, :], h2)
    out_ref[...] = out[:_N, :] + b2_ref[...]


def kernel(inputs, embeddings, W1, a_src1, a_dst1, b1, W2, a_src2, a_dst2,
           b2):
    del embeddings
    return pl.pallas_call(
        _fused_gat_kernel,
        out_shape=jax.ShapeDtypeStruct((_N, _UNITS), jnp.float32),
    )(inputs,
      W1.reshape(_F_IN, _HEADS * _UNITS),
      a_src1.reshape(1, _HEADS * _UNITS),
      a_dst1.reshape(1, _HEADS * _UNITS),
      b1.reshape(1, _HEADS * _UNITS),
      W2.reshape(_HEADS * _UNITS, _UNITS),
      a_src2, a_dst2,
      b2.reshape(1, _UNITS))


# speedup vs baseline: 7950.4131x; 1.0359x over previous
import jax
import jax.numpy as jnp
from jax.experimental import pallas as pl

_N = 1000
_NP = 1024
_F_IN = 64
_UNITS = 32
_HEADS = 4
_LOG2E = 1.4426950408889634


def _head_mix(nrows):
    r = jax.lax.broadcasted_iota(jnp.int32, (nrows, nrows // _UNITS), 0)
    c = jax.lax.broadcasted_iota(jnp.int32, (nrows, nrows // _UNITS), 1)
    return jnp.where(r // _UNITS == c, _LOG2E, 0.0).astype(jnp.float32)


def _fused_gat_kernel(x_ref, w1_ref, as1_ref, ad1_ref, b1_ref, w2_ref,
                      as2_ref, ad2_ref, b2_ref, out_ref):
    x = jnp.concatenate(
        [x_ref[...], jnp.zeros((_NP - _N, _F_IN), jnp.float32)], axis=0)
    h1 = jnp.dot(x, w1_ref[...],
                 preferred_element_type=jnp.float32)
    s1 = jnp.dot(
        jnp.concatenate([h1 * as1_ref[...], h1 * ad1_ref[...]], axis=1),
        _head_mix(2 * _HEADS * _UNITS),
        preferred_element_type=jnp.float32)
    s1t = s1.T

    col = jax.lax.broadcasted_iota(jnp.int32, (1, _NP), 1)
    neg_pad = jnp.where(col < _N, 0.0, -1e30).astype(jnp.float32)
    ones_col = jnp.full((_NP, 1), 1.0, jnp.float32)

    def _attend(st_col, ss_row, v):
        ss_m = ss_row + neg_pad
        ssmax = jnp.max(ss_m, axis=1, keepdims=True)
        m = st_col + ssmax
        m = jnp.maximum(m, 0.2 * m)
        st_m = st_col - m
        st2 = 0.2 * st_col - m
        ss2_m = 0.2 * ss_m
        e = jnp.maximum(st_m + ss_m, st2 + ss2_m)
        ex = jnp.exp2(e)
        va = jnp.concatenate([v, ones_col], axis=1)
        agg = jnp.dot(ex, va, preferred_element_type=jnp.float32)
        return agg[:, :_UNITS] / (agg[:, _UNITS:_UNITS + 1] + 1e-9)

    outs = []
    for h in range(_HEADS):
        outs.append(_attend(s1[:, h:h + 1],
                            s1t[_HEADS + h:_HEADS + h + 1, :],
                            h1[:, h * _UNITS:(h + 1) * _UNITS]))
    h1o = jnp.concatenate(outs, axis=1) + b1_ref[...]
    h1o = jnp.maximum(h1o, 0.0)

    h2 = jnp.dot(h1o, w2_ref[...],
                 preferred_element_type=jnp.float32)
    s2 = jnp.dot(
        jnp.concatenate([h2 * as2_ref[...], h2 * ad2_ref[...]], axis=1),
        _head_mix(2 * _UNITS),
        preferred_element_type=jnp.float32)
    s2t = s2.T
    out = _attend(s2[:, 0:1], s2t[1:2, :], h2)
    out_ref[...] = out[:_N, :] + b2_ref[...]


def kernel(inputs, embeddings, W1, a_src1, a_dst1, b1, W2, a_src2, a_dst2,
           b2):
    del embeddings
    return pl.pallas_call(
        _fused_gat_kernel,
        out_shape=jax.ShapeDtypeStruct((_N, _UNITS), jnp.float32),
    )(inputs,
      W1.reshape(_F_IN, _HEADS * _UNITS),
      a_src1.reshape(1, _HEADS * _UNITS),
      a_dst1.reshape(1, _HEADS * _UNITS),
      b1.reshape(1, _HEADS * _UNITS),
      W2.reshape(_HEADS * _UNITS, _UNITS),
      a_src2, a_dst2,
      b2.reshape(1, _UNITS))


# speedup vs baseline: 7954.4278x; 1.0005x over previous
import jax
import jax.numpy as jnp
from jax.experimental import pallas as pl

_N = 1000
_NP = 1024
_F_IN = 64
_UNITS = 32
_HEADS = 4
_VA = _UNITS + 1
_LOG2E = 1.4426950408889634


def _head_mix(nrows, ncols, group):
    r = jax.lax.broadcasted_iota(jnp.int32, (nrows, ncols), 0)
    c = jax.lax.broadcasted_iota(jnp.int32, (nrows, ncols), 1)
    return jnp.where(r // group == c, _LOG2E, 0.0).astype(jnp.float32)


def _ones_col_row(ncols):
    c = jax.lax.broadcasted_iota(jnp.int32, (1, ncols), 1)
    return jnp.where(c % _VA == _UNITS, 1.0, 0.0).astype(jnp.float32)


def _interleave_heads(w, nheads):
    rows = w.shape[0]
    z = jnp.zeros((rows, 1), jnp.float32)
    pieces = []
    for h in range(nheads):
        pieces.append(w[:, h * _UNITS:(h + 1) * _UNITS])
        pieces.append(z)
    return jnp.concatenate(pieces, axis=1)


def _fused_gat_kernel(x_ref, w1_ref, as1_ref, ad1_ref, b1_ref, w2_ref,
                      as2_ref, ad2_ref, b2_ref, out_ref):
    x = jnp.concatenate(
        [x_ref[...], jnp.full((_N, 1), 1.0, jnp.float32)], axis=1)
    x = jnp.concatenate(
        [x, jnp.zeros((_NP - _N, _F_IN + 1), jnp.float32)], axis=0)
    w1a = jnp.concatenate(
        [_interleave_heads(w1_ref[...], _HEADS), _ones_col_row(_HEADS * _VA)],
        axis=0)
    h1a = jnp.dot(x, w1a,
                  preferred_element_type=jnp.float32)
    mix1 = _head_mix(_HEADS * _VA, _HEADS, _VA)
    as1 = _interleave_heads(as1_ref[...], _HEADS)
    ad1 = _interleave_heads(ad1_ref[...], _HEADS)
    s_src = jnp.dot(h1a * as1, mix1,
                    preferred_element_type=jnp.float32)
    s_dst = jnp.dot(h1a * ad1, mix1,
                    preferred_element_type=jnp.float32)
    s_dst_t = s_dst.T

    col = jax.lax.broadcasted_iota(jnp.int32, (1, _NP), 1)
    neg_pad = jnp.where(col < _N, 0.0, -1e30).astype(jnp.float32)

    def _attend(st_col, ss_row, va):
        ss_m = ss_row + neg_pad
        ssmax = jnp.max(ss_m, axis=1, keepdims=True)
        z = st_col + ssmax
        m = jnp.maximum(z, 0.2 * z)
        d = ss_m - ssmax
        e1 = jnp.exp2(z - m)
        e2 = jnp.exp2(0.2 * z - m)
        f1 = jnp.exp2(d)
        f2 = jnp.exp2(0.2 * d)
        ex = jnp.maximum(e1 * f1, e2 * f2)
        agg = jnp.dot(ex, va, preferred_element_type=jnp.float32)
        recip = 1.0 / (agg[:, _UNITS:_VA] + 1e-9)
        return agg[:, :_UNITS] * recip

    outs = []
    for h in range(_HEADS):
        outs.append(_attend(s_src[:, h:h + 1],
                            s_dst_t[h:h + 1, :],
                            h1a[:, h * _VA:(h + 1) * _VA]))
    h1o = jnp.concatenate(outs, axis=1) + b1_ref[...]
    h1o = jnp.maximum(h1o, 0.0)
    h1o = jnp.concatenate(
        [h1o, jnp.full((_NP, 1), 1.0, jnp.float32)], axis=1)

    w2a = jnp.concatenate(
        [_interleave_heads(w2_ref[...], 1), _ones_col_row(_VA)], axis=0)
    h2a = jnp.dot(h1o, w2a,
                  preferred_element_type=jnp.float32)
    h2 = h2a[:, :_UNITS]
    l2e_col = jnp.full((_UNITS, 1), _LOG2E, jnp.float32)
    s2_src = jnp.dot(h2 * as2_ref[...], l2e_col,
                     preferred_element_type=jnp.float32)
    s2_dst = jnp.dot(h2 * ad2_ref[...], l2e_col,
                     preferred_element_type=jnp.float32)
    out = _attend(s2_src, s2_dst.T, h2a)
    out_ref[...] = out[:_N, :] + b2_ref[...]


def kernel(inputs, embeddings, W1, a_src1, a_dst1, b1, W2, a_src2, a_dst2,
           b2):
    del embeddings
    return pl.pallas_call(
        _fused_gat_kernel,
        out_shape=jax.ShapeDtypeStruct((_N, _UNITS), jnp.float32),
    )(inputs,
      W1.reshape(_F_IN, _HEADS * _UNITS),
      a_src1.reshape(1, _HEADS * _UNITS),
      a_dst1.reshape(1, _HEADS * _UNITS),
      b1.reshape(1, _HEADS * _UNITS),
      W2.reshape(_HEADS * _UNITS, _UNITS),
      a_src2, a_dst2,
      b2.reshape(1, _UNITS))


# speedup vs baseline: 8190.6477x; 1.0297x over previous
import jax
import jax.numpy as jnp
from jax.experimental import pallas as pl

_N = 1000
_NP = 1024
_F_IN = 64
_UNITS = 32
_HEADS = 4
_VA = _UNITS + 1
_LOG2E = 1.4426950408889634


def _head_mix(nrows, ncols, group):
    r = jax.lax.broadcasted_iota(jnp.int32, (nrows, ncols), 0)
    c = jax.lax.broadcasted_iota(jnp.int32, (nrows, ncols), 1)
    return jnp.where(r // group == c, _LOG2E, 0.0).astype(jnp.float32)


def _ones_col_row(ncols):
    c = jax.lax.broadcasted_iota(jnp.int32, (1, ncols), 1)
    return jnp.where(c % _VA == _UNITS, 1.0, 0.0).astype(jnp.float32)


def _interleave_heads(w, nheads):
    rows = w.shape[0]
    z = jnp.zeros((rows, 1), jnp.float32)
    pieces = []
    for h in range(nheads):
        pieces.append(w[:, h * _UNITS:(h + 1) * _UNITS])
        pieces.append(z)
    return jnp.concatenate(pieces, axis=1)


def _fused_gat_kernel(x_ref, w1_ref, as1_ref, ad1_ref, b1_ref, w2_ref,
                      as2_ref, ad2_ref, b2_ref, out_ref):
    x = jnp.concatenate(
        [x_ref[...], jnp.full((_N, 1), 1.0, jnp.float32)], axis=1)
    x = jnp.concatenate(
        [x, jnp.zeros((_NP - _N, _F_IN + 1), jnp.float32)], axis=0)
    w1a = jnp.concatenate(
        [_interleave_heads(w1_ref[...], _HEADS), _ones_col_row(_HEADS * _VA)],
        axis=0)
    h1a = jnp.dot(x, w1a,
                  preferred_element_type=jnp.float32)
    mix1 = _head_mix(_HEADS * _VA, _HEADS, _VA)
    as1 = _interleave_heads(as1_ref[...], _HEADS)
    ad1 = _interleave_heads(ad1_ref[...], _HEADS)
    s_src = jnp.dot(h1a * as1, mix1,
                    preferred_element_type=jnp.float32)
    s_dst = jnp.dot(h1a * ad1, mix1,
                    preferred_element_type=jnp.float32)
    s_dst_t = s_dst.T

    col = jax.lax.broadcasted_iota(jnp.int32, (1, _NP), 1)
    neg_pad = jnp.where(col < _N, 0.0, -1e30).astype(jnp.float32)

    def _attend(st_col, ss_row, va):
        ss_m = ss_row + neg_pad
        ssmax = jnp.max(ss_m, axis=1, keepdims=True)
        z = st_col + ssmax
        m = jnp.maximum(z, 0.2 * z)
        d = ss_m - ssmax
        e1 = jnp.exp2(z - m).astype(jnp.bfloat16)
        e2 = jnp.exp2(0.2 * z - m).astype(jnp.bfloat16)
        f1 = jnp.exp2(d).astype(jnp.bfloat16)
        f2 = jnp.exp2(0.2 * d).astype(jnp.bfloat16)
        ex = jnp.maximum(e1 * f1, e2 * f2)
        agg = jnp.dot(ex, va.astype(jnp.bfloat16),
                      preferred_element_type=jnp.float32)
        recip = 1.0 / (agg[:, _UNITS:_VA] + 1e-9)
        return agg[:, :_UNITS] * recip

    outs = []
    for h in range(_HEADS):
        outs.append(_attend(s_src[:, h:h + 1],
                            s_dst_t[h:h + 1, :],
                            h1a[:, h * _VA:(h + 1) * _VA]))
    h1o = jnp.concatenate(outs, axis=1) + b1_ref[...]
    h1o = jnp.maximum(h1o, 0.0)
    h1o = jnp.concatenate(
        [h1o, jnp.full((_NP, 1), 1.0, jnp.float32)], axis=1)

    w2a = jnp.concatenate(
        [_interleave_heads(w2_ref[...], 1), _ones_col_row(_VA)], axis=0)
    h2a = jnp.dot(h1o, w2a,
                  preferred_element_type=jnp.float32)
    h2 = h2a[:, :_UNITS]
    l2e_col = jnp.full((_UNITS, 1), _LOG2E, jnp.float32)
    s2_src = jnp.dot(h2 * as2_ref[...], l2e_col,
                     preferred_element_type=jnp.float32)
    s2_dst = jnp.dot(h2 * ad2_ref[...], l2e_col,
                     preferred_element_type=jnp.float32)
    out = _attend(s2_src, s2_dst.T, h2a)
    out_ref[...] = out[:_N, :] + b2_ref[...]


def kernel(inputs, embeddings, W1, a_src1, a_dst1, b1, W2, a_src2, a_dst2,
           b2):
    del embeddings
    return pl.pallas_call(
        _fused_gat_kernel,
        out_shape=jax.ShapeDtypeStruct((_N, _UNITS), jnp.float32),
    )(inputs,
      W1.reshape(_F_IN, _HEADS * _UNITS),
      a_src1.reshape(1, _HEADS * _UNITS),
      a_dst1.reshape(1, _HEADS * _UNITS),
      b1.reshape(1, _HEADS * _UNITS),
      W2.reshape(_HEADS * _UNITS, _UNITS),
      a_src2, a_dst2,
      b2.reshape(1, _UNITS))


# speedup vs baseline: 8192.9131x; 1.0003x over previous
import jax
import jax.numpy as jnp
from jax.experimental import pallas as pl

_N = 1000
_NP = 1024
_F_IN = 64
_UNITS = 32
_HEADS = 4
_VA = _UNITS + 1
_LOG2E = 1.4426950408889634


def _head_mix(nrows, ncols, group):
    r = jax.lax.broadcasted_iota(jnp.int32, (nrows, ncols), 0)
    c = jax.lax.broadcasted_iota(jnp.int32, (nrows, ncols), 1)
    return jnp.where(r // group == c, _LOG2E, 0.0).astype(jnp.float32)


def _ones_col_row(ncols):
    c = jax.lax.broadcasted_iota(jnp.int32, (1, ncols), 1)
    return jnp.where(c % _VA == _UNITS, 1.0, 0.0).astype(jnp.float32)


def _interleave_heads(w, nheads):
    rows = w.shape[0]
    z = jnp.zeros((rows, 1), jnp.float32)
    pieces = []
    for h in range(nheads):
        pieces.append(w[:, h * _UNITS:(h + 1) * _UNITS])
        pieces.append(z)
    return jnp.concatenate(pieces, axis=1)


def _fused_gat_kernel(x_ref, w1_ref, as1_ref, ad1_ref, b1_ref, w2_ref,
                      as2_ref, ad2_ref, b2_ref, out_ref):
    x = jnp.concatenate(
        [x_ref[...], jnp.full((_N, 1), 1.0, jnp.float32)], axis=1)
    x = jnp.concatenate(
        [x, jnp.zeros((_NP - _N, _F_IN + 1), jnp.float32)], axis=0)
    w1a = jnp.concatenate(
        [_interleave_heads(w1_ref[...], _HEADS), _ones_col_row(_HEADS * _VA)],
        axis=0)
    h1a = jnp.dot(x, w1a,
                  preferred_element_type=jnp.float32)
    mix1 = _head_mix(_HEADS * _VA, _HEADS, _VA)
    as1 = _interleave_heads(as1_ref[...], _HEADS)
    ad1 = _interleave_heads(ad1_ref[...], _HEADS)
    s_src = jnp.dot(h1a * as1, mix1,
                    preferred_element_type=jnp.float32)
    s_dst = jnp.dot(h1a * ad1, mix1,
                    preferred_element_type=jnp.float32)
    s_dst_t = s_dst.T

    col = jax.lax.broadcasted_iota(jnp.int32, (1, _NP), 1)
    neg_pad = jnp.where(col < _N, 0.0, -1e30).astype(jnp.float32)

    def _attend(st_col, ss_row, va):
        ss_m = ss_row + neg_pad
        ssmax = jnp.max(ss_m, axis=1, keepdims=True)
        z = st_col + ssmax
        m = jnp.maximum(z, 0.2 * z)
        d = ss_m - ssmax
        e1 = jnp.exp2(z - m).astype(jnp.bfloat16)
        e2 = jnp.exp2(0.2 * z - m).astype(jnp.bfloat16)
        f1 = jnp.exp2(d).astype(jnp.bfloat16)
        f2 = jnp.exp2(0.2 * d).astype(jnp.bfloat16)
        ex = jnp.maximum(e1 * f1, e2 * f2)
        agg = jnp.dot(ex, va.astype(jnp.bfloat16),
                      preferred_element_type=jnp.float32)
        recip = 1.0 / (agg[:, _UNITS:_VA] + 1e-9)
        return agg[:, :_UNITS] * recip

    outs = []
    for h in range(_HEADS):
        outs.append(_attend(s_src[:, h:h + 1],
                            s_dst_t[h:h + 1, :],
                            h1a[:, h * _VA:(h + 1) * _VA]))
    h1o = jnp.concatenate(outs, axis=1) + b1_ref[...]
    h1o = jnp.maximum(h1o, 0.0)
    h1o = jnp.concatenate(
        [h1o, jnp.full((_NP, 1), 1.0, jnp.float32)], axis=1)

    w2a = jnp.concatenate(
        [_interleave_heads(w2_ref[...], 1), _ones_col_row(_VA)], axis=0)
    h2a = jnp.dot(h1o, w2a,
                  preferred_element_type=jnp.float32)
    h2 = h2a[:, :_UNITS]
    s2_src = jnp.dot(h2, as2_ref[...].T * _LOG2E,
                     preferred_element_type=jnp.float32)
    s2_dst = jnp.dot(h2, ad2_ref[...].T * _LOG2E,
                     preferred_element_type=jnp.float32)
    out = _attend(s2_src, s2_dst.T, h2a)
    out_ref[...] = out[:_N, :] + b2_ref[...]


def kernel(inputs, embeddings, W1, a_src1, a_dst1, b1, W2, a_src2, a_dst2,
           b2):
    del embeddings
    return pl.pallas_call(
        _fused_gat_kernel,
        out_shape=jax.ShapeDtypeStruct((_N, _UNITS), jnp.float32),
    )(inputs,
      W1.reshape(_F_IN, _HEADS * _UNITS),
      a_src1.reshape(1, _HEADS * _UNITS),
      a_dst1.reshape(1, _HEADS * _UNITS),
      b1.reshape(1, _HEADS * _UNITS),
      W2.reshape(_HEADS * _UNITS, _UNITS),
      a_src2, a_dst2,
      b2.reshape(1, _UNITS))
